# Initial kernel scaffold; baseline (speedup 1.0000x reference)
import functools
import jax
import jax.numpy as jnp
from jax import lax
from jax.experimental import pallas as pl
from jax.experimental.pallas import tpu as pltpu
from jax.experimental.pallas import tpu_sc as plsc

NW = 32
_F32 = jnp.float32


def _pe_table(n, d):
    position = jnp.arange(0, n, dtype=_F32).reshape(-1, 1)
    div_term = 10000.0 ** (jnp.arange(0, d, 2, dtype=_F32) / d)
    pe = jnp.zeros((n, d), dtype=_F32)
    pe = pe.at[:, 0::2].set(jnp.sin(position * div_term))
    pe = pe.at[:, 1::2].set(jnp.cos(position * div_term))
    return pe


def _gelu16(x):
    u = x * 0.7071067811865476
    s = jnp.sign(u)
    a = jnp.abs(u)
    t = 1.0 / (1.0 + 0.3275911 * a)
    poly = ((((1.061405429 * t - 1.453152027) * t + 1.421413741) * t
             - 0.284496736) * t + 0.254829592) * t
    erf = s * (1.0 - poly * jnp.exp(-a * a))
    return 0.5 * x * (1.0 + erf)


def _make_sc_energies(B, T, K, H, P, V):
    NB = B // NW
    TK = T * K
    NE2 = NB * T * (K - 1)
    CH = 8
    NCH = NE2 // CH
    HH = H * H
    NQ = H // 16

    mesh = plsc.VectorSubcoreMesh(core_axis_name="c", subcore_axis_name="s")

    @functools.partial(
        pl.kernel,
        out_type=jax.ShapeDtypeStruct((B, TK), _F32),
        mesh=mesh,
        scratch_types=[
            pltpu.VMEM((NB, TK), jnp.int32),
            pltpu.VMEM((NB, TK), jnp.int32),
            pltpu.VMEM((NB, TK), jnp.int32),
            pltpu.VMEM((T, NB), jnp.int32),
            pltpu.VMEM((NE2 // 8, 8), jnp.int32),
            pltpu.VMEM((16,), jnp.int32),
            pltpu.VMEM((NB, T, H), _F32),
            pltpu.VMEM((NB, T, H), _F32),
            pltpu.VMEM((NB, TK), _F32),
            pltpu.VMEM((T + 1, H), _F32),
            pltpu.VMEM((T, T), _F32),
            pltpu.VMEM((2, CH, HH), _F32),
            pltpu.VMEM((2, CH, H), _F32),
            pltpu.VMEM((16, H), _F32),
            pltpu.SemaphoreType.DMA,
            pltpu.SemaphoreType.DMA,
            pltpu.SemaphoreType.DMA,
        ],
    )
    def sc_energies(src_h, tgt_h, w_h, b_h, nb_h, pe_h, wt_h, out_h,
                    srcv, tgtv, pidxv, idx1v, idx2v, idx0v, ev, ov, env,
                    pev, wtv, wbufv, bbufv, nbbufv, sem0, sem1, semn):
        sems = (sem0, sem1)
        wid = lax.axis_index("s") * 2 + lax.axis_index("c")
        b0 = wid * NB

        pltpu.sync_copy(src_h.at[pl.ds(b0, NB)], srcv)
        pltpu.sync_copy(tgt_h.at[pl.ds(b0, NB)], tgtv)
        pltpu.sync_copy(pe_h, pev)
        pltpu.sync_copy(wt_h, wtv)

        lanes = lax.iota(jnp.int32, 16)

        for b in range(NB):
            def pidx_body(c, _, b=b):
                s = srcv[b, pl.ds(c * 16, 16)]
                t = tgtv[b, pl.ds(c * 16, 16)]
                pidxv[b, pl.ds(c * 16, 16)] = (s * 31 + t * 17) % P
                return 0
            lax.fori_loop(0, TK // 16, pidx_body, 0)

        def i1_body(t, _):
            for b in range(NB):
                idx1v[t, b] = pidxv[b, t * K]
            return 0
        lax.fori_loop(0, T, i1_body, 0)

        def i2_body(c, _):
            e2 = c * 16 + lanes
            b = e2 // (T * (K - 1))
            r = e2 - b * (T * (K - 1))
            t = r // (K - 1)
            k = r - t * (K - 1) + 1
            vals = plsc.load_gather(pidxv, [b, t * K + k])
            plsc.store_scatter(idx2v, [e2 // 8, e2 % 8], vals)
            return 0
        lax.fori_loop(0, NE2 // 16, i2_body, 0)

        bl = jnp.where(lanes < NB, lanes, 0)
        idx0v[...] = plsc.load_gather(srcv, [bl, jnp.zeros((16,), jnp.int32)])
        nb_cp = pltpu.async_copy(nb_h.at[idx0v], nbbufv, semn)

        def dma_start(idxref, r, slot):
            pltpu.async_copy(w_h.at[idxref.at[r]], wbufv.at[slot], sems[slot])
            pltpu.async_copy(b_h.at[idxref.at[r]], bbufv.at[slot], sems[slot])

        def dma_wait(idxref, r, slot):
            pltpu.make_async_copy(
                w_h.at[idxref.at[r]], wbufv.at[slot], sems[slot]).wait()
            pltpu.make_async_copy(
                b_h.at[idxref.at[r]], bbufv.at[slot], sems[slot]).wait()

        def sumsq(accs):
            tv = accs[0] * accs[0]
            for q in range(1, NQ):
                tv = tv + accs[q] * accs[q]
            return jnp.sum(tv)

        def mat_gelu(slot, j, b, t):
            accs = tuple(
                bbufv[slot, j, pl.ds(q * 16, 16)]
                + pev[t + 1, pl.ds(q * 16, 16)]
                for q in range(NQ))

            def istep(i8, acc):
                acc = list(acc)
                for u in range(8):
                    i = i8 * 8 + u
                    sv = jnp.full((16,), ev[b, t, i])
                    for q in range(NQ):
                        acc[q] = acc[q] + sv * wbufv[
                            slot, j, pl.ds(i * H + q * 16, 16)]
                return tuple(acc)

            accs = lax.fori_loop(0, H // 8, istep, accs)
            return tuple(_gelu16(a) for a in accs)

        nb_cp.wait()
        for b in range(NB):
            for q in range(NQ):
                x = (1.0 / H + nbbufv[b, pl.ds(q * 16, 16)]
                     + pev[0, pl.ds(q * 16, 16)])
                ev[b, 0, pl.ds(q * 16, 16)] = _gelu16(x)

        dma_start(idx1v, 0, 0)
        dma_start(idx1v, 1, 1)

        def p1_body(t2, _):
            for slot in range(2):
                t = t2 * 2 + slot
                dma_wait(idx1v, t, slot)

                @pl.when(t > 0)
                def _():
                    def jstep(j, carry):
                        sv = jnp.full((16,), wtv[t, j])
                        out = []
                        for b in range(NB):
                            for q in range(NQ):
                                out.append(carry[b * NQ + q]
                                           + sv * ov[b, j, pl.ds(q * 16, 16)])
                        return tuple(out)
                    init = tuple(jnp.zeros((16,), _F32)
                                 for _ in range(NB * NQ))
                    res = lax.fori_loop(0, t, jstep, init)
                    for b in range(NB):
                        for q in range(NQ):
                            ev[b, t, pl.ds(q * 16, 16)] = res[b * NQ + q]

                def bstep(b, _):
                    accs = mat_gelu(slot, b, b, t)
                    for q in range(NQ):
                        ov[b, t, pl.ds(q * 16, 16)] = accs[q]
                    env[b, t * K] = sumsq(accs)
                    return 0
                lax.fori_loop(0, NB, bstep, 0)

                @pl.when(t + 2 < T)
                def _():
                    dma_start(idx1v, t + 2, slot)
            return 0
        lax.fori_loop(0, T // 2, p1_body, 0)

        dma_start(idx2v, 0, 0)
        dma_start(idx2v, 1, 1)

        def p2_body(g2, _):
            for slot in range(2):
                g = g2 * 2 + slot
                dma_wait(idx2v, g, slot)

                def jstep(j, _):
                    e2 = g * CH + j
                    b = e2 // (T * (K - 1))
                    r = e2 - b * (T * (K - 1))
                    t = r // (K - 1)
                    k = r - t * (K - 1) + 1
                    accs = mat_gelu(slot, j, b, t)
                    env[b, t * K + k] = sumsq(accs)
                    return 0
                lax.fori_loop(0, CH, jstep, 0)

                @pl.when(g + 2 < NCH)
                def _():
                    dma_start(idx2v, g + 2, slot)
            return 0
        lax.fori_loop(0, NCH // 2, p2_body, 0)

        pltpu.sync_copy(env, out_h.at[pl.ds(b0, NB)])

    return sc_energies


def _loss_tc(sumsq2, BT):
    def body(x_ref, o_ref):
        e = jnp.sqrt(x_ref[...])
        m = jnp.max(e, axis=1, keepdims=True)
        lse = m[:, 0] + jnp.log(jnp.sum(jnp.exp(e - m), axis=1))
        o_ref[0, 0] = jnp.sum(lse - e[:, 0]) / _F32(BT)

    out = pl.pallas_call(
        body,
        out_shape=jax.ShapeDtypeStruct((1, 1), _F32),
        out_specs=pl.BlockSpec(memory_space=pltpu.MemorySpace.SMEM),
    )(sumsq2)
    return out[0, 0]


@jax.jit
def kernel(neighbor_ids, weights, biases, node_bias, positions):
    B, T, K, _ = neighbor_ids.shape
    P, H, _ = weights.shape
    V = node_bias.shape[0]

    src = neighbor_ids[..., 0].reshape(B, T * K).astype(jnp.int32)
    tgt = neighbor_ids[..., 1].reshape(B, T * K).astype(jnp.int32)
    w2d = weights.reshape(P, H * H)
    b2d = biases.reshape(P, H)
    nb2d = node_bias.reshape(V, H)
    pe = _pe_table(T + 1, H)

    pos = positions[0, :T, 0]
    iot = jnp.arange(T)
    valid = iot[None, :] < iot[:, None]
    mx = jnp.max(jnp.where(valid, pos[None, :], -jnp.inf), axis=1,
                 keepdims=True, initial=-jnp.inf)
    ex = jnp.where(valid, jnp.exp(pos[None, :] - mx), 0.0)
    den = jnp.maximum(jnp.sum(ex, axis=1, keepdims=True), 1e-30)
    wt = (ex / den).astype(_F32)

    sc = _make_sc_energies(B, T, K, H, P, V)
    sumsq = sc(src, tgt, w2d, b2d, nb2d, pe, wt)
    return _loss_tc(sumsq.reshape(B * T, K), B * T)


# speedup vs baseline: 2.5014x; 2.5014x over previous
import functools
import jax
import jax.numpy as jnp
from jax import lax
from jax.experimental import pallas as pl
from jax.experimental.pallas import tpu as pltpu
from jax.experimental.pallas import tpu_sc as plsc

NW = 32
_F32 = jnp.float32
_I32 = jnp.int32


def _pe_table(n, d):
    position = jnp.arange(0, n, dtype=_F32).reshape(-1, 1)
    div_term = 10000.0 ** (jnp.arange(0, d, 2, dtype=_F32) / d)
    pe = jnp.zeros((n, d), dtype=_F32)
    pe = pe.at[:, 0::2].set(jnp.sin(position * div_term))
    pe = pe.at[:, 1::2].set(jnp.cos(position * div_term))
    return pe


def _gelu16(x):
    u = x * 0.7071067811865476
    s = jnp.sign(u)
    a = jnp.abs(u)
    t = 1.0 / (1.0 + 0.3275911 * a)
    poly = ((((1.061405429 * t - 1.453152027) * t + 1.421413741) * t
             - 0.284496736) * t + 0.254829592) * t
    erf = s * (1.0 - poly * jnp.exp(-a * a))
    return 0.5 * x * (1.0 + erf)


def _make_sc_energies(B, T, K, H, P, V):
    NB = B // NW
    TK = T * K
    CH = K - 1
    NCH = NB * T
    HH = H * H
    NQ = H // 16

    mesh = plsc.VectorSubcoreMesh(core_axis_name="c", subcore_axis_name="s")

    @functools.partial(
        pl.kernel,
        out_type=jax.ShapeDtypeStruct((B, TK), _F32),
        mesh=mesh,
        compiler_params=pltpu.CompilerParams(
            needs_layout_passes=False, use_tc_tiling_on_sc=False),
        scratch_types=[
            pltpu.VMEM((NB, TK), _I32),
            pltpu.VMEM((NB, TK), _I32),
            pltpu.VMEM((NB, TK), _I32),
            pltpu.VMEM((T, NB), _I32),
            pltpu.VMEM((NCH, CH), _I32),
            pltpu.VMEM((16,), _I32),
            pltpu.VMEM((NB, T, H), _F32),
            pltpu.VMEM((NB, T, H), _F32),
            pltpu.VMEM((NB, TK), _F32),
            pltpu.VMEM((T + 1, H), _F32),
            pltpu.VMEM((T, T), _F32),
            pltpu.VMEM((2, NB, HH), _F32),
            pltpu.VMEM((2, NB, H), _F32),
            pltpu.VMEM((16, H), _F32),
            pltpu.SemaphoreType.DMA,
            pltpu.SemaphoreType.DMA,
            pltpu.SemaphoreType.DMA,
        ],
    )
    def sc_energies(src_h, tgt_h, w_h, b_h, nb_h, pe_h, wt_h, out_h,
                    srcv, tgtv, pidxv, idx1v, idx2v, idx0v, ev, ov, env,
                    pev, wtv, wbufv, bbufv, nbbufv, sem0, sem1, semn):
        sems = (sem0, sem1)
        wid = lax.axis_index("s") * 2 + lax.axis_index("c")
        b0 = wid * NB

        pltpu.sync_copy(src_h.at[pl.ds(b0, NB)], srcv)
        pltpu.sync_copy(tgt_h.at[pl.ds(b0, NB)], tgtv)
        pltpu.sync_copy(pe_h, pev)
        pltpu.sync_copy(wt_h, wtv)

        lanes = lax.iota(_I32, 16)
        lmask7 = lanes < CH

        for b in range(NB):
            def pidx_body(c, _, b=b):
                s = srcv[b, pl.ds(c * 16, 16)]
                t = tgtv[b, pl.ds(c * 16, 16)]
                pidxv[b, pl.ds(c * 16, 16)] = (s * 31 + t * 17) % P
                return 0
            lax.fori_loop(0, TK // 16, pidx_body, 0)

        def i1_body(c, _):
            p = c * 16 + lanes
            tt = p // NB
            bb = p - tt * NB
            vals = plsc.load_gather(pidxv, [bb, tt * K])
            plsc.store_scatter(idx1v, [tt, bb], vals)
            return 0
        lax.fori_loop(0, T * NB // 16, i1_body, 0)

        def i2_body(g, _):
            b = g // T
            t = g - b * T
            cols = t * K + 1 + lanes
            vals = plsc.load_gather(
                pidxv, [jnp.full((16,), b, _I32),
                        jnp.where(lmask7, cols, 0)])
            plsc.store_scatter(idx2v, [jnp.full((16,), g, _I32), lanes],
                               vals, mask=lmask7)
            return 0
        lax.fori_loop(0, NCH, i2_body, 0)

        bl = jnp.where(lanes < NB, lanes, 0)
        idx0v[...] = plsc.load_gather(srcv, [bl, jnp.zeros((16,), _I32)])
        nb_cp = pltpu.async_copy(nb_h.at[idx0v], nbbufv, semn)

        def dma_start(idxref, r, slot, n):
            pltpu.async_copy(w_h.at[idxref.at[r]],
                             wbufv.at[slot, pl.ds(0, n)], sems[slot])
            pltpu.async_copy(b_h.at[idxref.at[r]],
                             bbufv.at[slot, pl.ds(0, n)], sems[slot])

        def dma_wait(idxref, r, slot, n):
            pltpu.make_async_copy(
                w_h.at[idxref.at[r]],
                wbufv.at[slot, pl.ds(0, n)], sems[slot]).wait()
            pltpu.make_async_copy(
                b_h.at[idxref.at[r]],
                bbufv.at[slot, pl.ds(0, n)], sems[slot]).wait()

        def sumsq(accs):
            tv = accs[0] * accs[0]
            for q in range(1, NQ):
                tv = tv + accs[q] * accs[q]
            return jnp.sum(tv)

        lane0 = lanes == 0

        def store_scalar(ref, row, col, val):
            plsc.store_scatter(ref,
                               [jnp.full((16,), row, _I32),
                                jnp.full((16,), col, _I32)],
                               jnp.full((16,), val), mask=lane0)

        def mat_gelu(slot, j, evs, t):
            accs = tuple(
                bbufv[slot, j, pl.ds(q * 16, 16)]
                + pev[t + 1, pl.ds(q * 16, 16)]
                for q in range(NQ))
            acc = list(accs)
            for i in range(H):
                sv = evs[i // 16].at[jnp.full((16,), i % 16, _I32)].get(
                    mode="promise_in_bounds")
                for q in range(NQ):
                    acc[q] = acc[q] + sv * wbufv[
                        slot, j, pl.ds(i * H + q * 16, 16)]
            return tuple(_gelu16(a) for a in acc)

        def load_evs(b, t):
            return tuple(ev[b, t, pl.ds(q * 16, 16)] for q in range(NQ))

        nb_cp.wait()
        for b in range(NB):
            for q in range(NQ):
                x = (1.0 / H + nbbufv[b, pl.ds(q * 16, 16)]
                     + pev[0, pl.ds(q * 16, 16)])
                ev[b, 0, pl.ds(q * 16, 16)] = _gelu16(x)

        dma_start(idx1v, 0, 0, NB)
        dma_start(idx1v, 1, 1, NB)

        def p1_body(t2, _):
            for slot in range(2):
                t = t2 * 2 + slot
                dma_wait(idx1v, t, slot, NB)

                @pl.when(t > 0)
                def _():
                    def jstep(j, carry):
                        sv = plsc.load_gather(
                            wtv, [jnp.full((16,), t, _I32),
                                  jnp.full((16,), j, _I32)])
                        out = []
                        for b in range(NB):
                            for q in range(NQ):
                                out.append(carry[b * NQ + q]
                                           + sv * ov[b, j, pl.ds(q * 16, 16)])
                        return tuple(out)
                    init = tuple(jnp.zeros((16,), _F32)
                                 for _ in range(NB * NQ))
                    res = lax.fori_loop(0, t, jstep, init)
                    for b in range(NB):
                        for q in range(NQ):
                            ev[b, t, pl.ds(q * 16, 16)] = res[b * NQ + q]

                def bstep(b, _):
                    accs = mat_gelu(slot, b, load_evs(b, t), t)
                    for q in range(NQ):
                        ov[b, t, pl.ds(q * 16, 16)] = accs[q]
                    store_scalar(env, b, t * K, sumsq(accs))
                    return 0
                lax.fori_loop(0, NB, bstep, 0)

                @pl.when(t + 2 < T)
                def _():
                    dma_start(idx1v, t + 2, slot, NB)
            return 0
        lax.fori_loop(0, T // 2, p1_body, 0)

        dma_start(idx2v, 0, 0, CH)
        dma_start(idx2v, 1, 1, CH)

        def p2_body(g2, _):
            for slot in range(2):
                g = g2 * 2 + slot
                dma_wait(idx2v, g, slot, CH)
                b = g // T
                t = g - b * T
                evs = load_evs(b, t)

                def jstep(j, _):
                    accs = mat_gelu(slot, j, evs, t)
                    store_scalar(env, b, t * K + 1 + j, sumsq(accs))
                    return 0
                lax.fori_loop(0, CH, jstep, 0)

                @pl.when(g + 2 < NCH)
                def _():
                    dma_start(idx2v, g + 2, slot, CH)
            return 0
        lax.fori_loop(0, NCH // 2, p2_body, 0)

        pltpu.sync_copy(env, out_h.at[pl.ds(b0, NB)])

    return sc_energies


def _loss_tc(sumsq2, BT):
    def body(x_ref, o_ref):
        e = jnp.sqrt(x_ref[...])
        m = jnp.max(e, axis=1, keepdims=True)
        lse = m[:, 0] + jnp.log(jnp.sum(jnp.exp(e - m), axis=1))
        o_ref[0, 0] = jnp.sum(lse - e[:, 0]) / _F32(BT)

    out = pl.pallas_call(
        body,
        out_shape=jax.ShapeDtypeStruct((1, 1), _F32),
        out_specs=pl.BlockSpec(memory_space=pltpu.MemorySpace.SMEM),
    )(sumsq2)
    return out[0, 0]


@jax.jit
def kernel(neighbor_ids, weights, biases, node_bias, positions):
    B, T, K, _ = neighbor_ids.shape
    P, H, _ = weights.shape
    V = node_bias.shape[0]

    src = neighbor_ids[..., 0].reshape(B, T * K).astype(_I32)
    tgt = neighbor_ids[..., 1].reshape(B, T * K).astype(_I32)
    w2d = weights.reshape(P, H * H)
    b2d = biases.reshape(P, H)
    nb2d = node_bias.reshape(V, H)
    pe = _pe_table(T + 1, H)

    pos = positions[0, :T, 0]
    iot = jnp.arange(T)
    valid = iot[None, :] < iot[:, None]
    mx = jnp.max(jnp.where(valid, pos[None, :], -jnp.inf), axis=1,
                 keepdims=True, initial=-jnp.inf)
    ex = jnp.where(valid, jnp.exp(pos[None, :] - mx), 0.0)
    den = jnp.maximum(jnp.sum(ex, axis=1, keepdims=True), 1e-30)
    wt = (ex / den).astype(_F32)

    sc = _make_sc_energies(B, T, K, H, P, V)
    sumsq = sc(src, tgt, w2d, b2d, nb2d, pe, wt)
    return _loss_tc(sumsq.reshape(B * T, K), B * T)


# speedup vs baseline: 2.8088x; 1.1229x over previous
import functools
import jax
import jax.numpy as jnp
from jax import lax
from jax.experimental import pallas as pl
from jax.experimental.pallas import tpu as pltpu
from jax.experimental.pallas import tpu_sc as plsc

NW = 32
_F32 = jnp.float32
_I32 = jnp.int32


def _pe_table(n, d):
    position = jnp.arange(0, n, dtype=_F32).reshape(-1, 1)
    div_term = 10000.0 ** (jnp.arange(0, d, 2, dtype=_F32) / d)
    pe = jnp.zeros((n, d), dtype=_F32)
    pe = pe.at[:, 0::2].set(jnp.sin(position * div_term))
    pe = pe.at[:, 1::2].set(jnp.cos(position * div_term))
    return pe


def _gelu16(x):
    u = x * 0.7071067811865476
    s = jnp.sign(u)
    a = jnp.abs(u)
    t = 1.0 / (1.0 + 0.3275911 * a)
    poly = ((((1.061405429 * t - 1.453152027) * t + 1.421413741) * t
             - 0.284496736) * t + 0.254829592) * t
    erf = s * (1.0 - poly * jnp.exp(-a * a))
    return 0.5 * x * (1.0 + erf)


def _make_sc_energies(B, T, K, H, P, V):
    NB = B // NW
    TK = T * K
    CH = K - 1
    NCH = NB * T
    HH = H * H
    NQ = H // 16

    mesh = plsc.VectorSubcoreMesh(core_axis_name="c", subcore_axis_name="s")

    @functools.partial(
        pl.kernel,
        out_type=jax.ShapeDtypeStruct((B, TK), _F32),
        mesh=mesh,
        compiler_params=pltpu.CompilerParams(
            needs_layout_passes=False, use_tc_tiling_on_sc=False),
        scratch_types=[
            pltpu.VMEM((NB, TK), _I32),
            pltpu.VMEM((NB, TK), _I32),
            pltpu.VMEM((T, NB), _I32),
            pltpu.VMEM((NCH, CH), _I32),
            pltpu.VMEM((16,), _I32),
            pltpu.VMEM((NB, T, H), _F32),
            pltpu.VMEM((NB, H), _F32),
            pltpu.VMEM((NB, TK), _F32),
            pltpu.VMEM((T + 1, H), _F32),
            pltpu.VMEM((2, T), _F32),
            pltpu.VMEM((3, NB, HH), _F32),
            pltpu.VMEM((3, NB, H), _F32),
            pltpu.VMEM((16, H), _F32),
            pltpu.SemaphoreType.DMA,
            pltpu.SemaphoreType.DMA,
            pltpu.SemaphoreType.DMA,
            pltpu.SemaphoreType.DMA,
        ],
    )
    def sc_energies(src_h, tgt_h, w_h, b_h, nb_h, pe_h, ewt_h, out_h,
                    srcv, pidxv, idx1v, idx2v, idx0v, ev, av, env,
                    pev, ewt, wbufv, bbufv, nbbufv, sem0, sem1, sem2, semn):
        sems = (sem0, sem1, sem2)
        wid = lax.axis_index("s") * 2 + lax.axis_index("c")
        b0 = wid * NB

        pltpu.sync_copy(src_h.at[pl.ds(b0, NB)], srcv)
        pltpu.sync_copy(tgt_h.at[pl.ds(b0, NB)], pidxv)
        pltpu.sync_copy(pe_h, pev)
        pltpu.sync_copy(ewt_h, ewt)

        lanes = lax.iota(_I32, 16)
        lmask7 = lanes < CH

        for b in range(NB):
            def pidx_body(c, _, b=b):
                s = srcv[b, pl.ds(c * 16, 16)]
                t = pidxv[b, pl.ds(c * 16, 16)]
                pidxv[b, pl.ds(c * 16, 16)] = (s * 31 + t * 17) % P
                return 0
            lax.fori_loop(0, TK // 16, pidx_body, 0)

        def i1_body(c, _):
            p = c * 16 + lanes
            tt = p // NB
            bb = p - tt * NB
            vals = plsc.load_gather(pidxv, [bb, tt * K])
            plsc.store_scatter(idx1v, [tt, bb], vals)
            return 0
        lax.fori_loop(0, T * NB // 16, i1_body, 0)

        def i2_body(g, _):
            b = g // T
            t = g - b * T
            cols = t * K + 1 + lanes
            vals = plsc.load_gather(
                pidxv, [jnp.full((16,), b, _I32),
                        jnp.where(lmask7, cols, 0)])
            plsc.store_scatter(idx2v, [jnp.full((16,), g, _I32), lanes],
                               vals, mask=lmask7)
            return 0
        lax.fori_loop(0, NCH, i2_body, 0)

        bl = jnp.where(lanes < NB, lanes, 0)
        idx0v[...] = plsc.load_gather(srcv, [bl, jnp.zeros((16,), _I32)])
        nb_cp = pltpu.async_copy(nb_h.at[idx0v], nbbufv, semn)

        def dma_start(idxref, r, slot, n):
            pltpu.async_copy(w_h.at[idxref.at[r]],
                             wbufv.at[slot, pl.ds(0, n)], sems[slot])
            pltpu.async_copy(b_h.at[idxref.at[r]],
                             bbufv.at[slot, pl.ds(0, n)], sems[slot])

        def dma_wait(idxref, r, slot, n):
            pltpu.make_async_copy(
                w_h.at[idxref.at[r]],
                wbufv.at[slot, pl.ds(0, n)], sems[slot]).wait()
            pltpu.make_async_copy(
                b_h.at[idxref.at[r]],
                bbufv.at[slot, pl.ds(0, n)], sems[slot]).wait()

        def sumsq(accs):
            tv = accs[0] * accs[0]
            for q in range(1, NQ):
                tv = tv + accs[q] * accs[q]
            return jnp.sum(tv)

        lane0 = lanes == 0

        def store_scalar(ref, row, col, val):
            plsc.store_scatter(ref,
                               [jnp.full((16,), row, _I32),
                                jnp.full((16,), col, _I32)],
                               jnp.full((16,), val), mask=lane0)

        def mat_gelu(slot, j, evs, t):
            acc = tuple(
                bbufv[slot, j, pl.ds(q * 16, 16)]
                + pev[t + 1, pl.ds(q * 16, 16)]
                for q in range(NQ))

            def mk_step(q_e):
                def mstep(m4, a):
                    a = list(a)
                    for u in range(4):
                        m = m4 * 4 + u
                        sv = evs[q_e].at[jnp.full((16,), m, _I32)].get(
                            mode="promise_in_bounds")
                        off = (q_e * 16 + m) * H
                        for q in range(NQ):
                            a[q] = a[q] + sv * wbufv[
                                slot, j, pl.ds(off + q * 16, 16)]
                    return tuple(a)
                return mstep

            for q_e in range(NQ):
                acc = lax.fori_loop(0, 4, mk_step(q_e), acc)
            return tuple(_gelu16(a) for a in acc)

        nb_cp.wait()
        for b in range(NB):
            for q in range(NQ):
                x = (1.0 / H + nbbufv[b, pl.ds(q * 16, 16)]
                     + pev[0, pl.ds(q * 16, 16)])
                ev[b, 0, pl.ds(q * 16, 16)] = _gelu16(x)
                av[b, pl.ds(q * 16, 16)] = jnp.zeros((16,), _F32)

        dma_start(idx1v, 0, 0, NB)
        dma_start(idx1v, 1, 1, NB)

        def p1_body(t2, _):
            for slot in range(2):
                t = t2 * 2 + slot
                dma_wait(idx1v, t, slot, NB)
                tpos = jnp.full((16,), t, _I32) > 0
                inv_s = plsc.load_gather(
                    ewt, [jnp.full((16,), 1, _I32),
                          jnp.full((16,), t, _I32)])
                ew_t = plsc.load_gather(
                    ewt, [jnp.full((16,), 0, _I32),
                          jnp.full((16,), t, _I32)])

                def bstep(b, _):
                    A = tuple(av[b, pl.ds(q * 16, 16)] for q in range(NQ))
                    evs = tuple(
                        jnp.where(tpos, A[q] * inv_s,
                                  ev[b, t, pl.ds(q * 16, 16)])
                        for q in range(NQ))
                    for q in range(NQ):
                        ev[b, t, pl.ds(q * 16, 16)] = evs[q]
                    accs = mat_gelu(slot, b, evs, t)
                    for q in range(NQ):
                        av[b, pl.ds(q * 16, 16)] = A[q] + ew_t * accs[q]
                    store_scalar(env, b, t * K, sumsq(accs))
                    return 0
                lax.fori_loop(0, NB, bstep, 0)

                @pl.when(t + 2 < T)
                def _():
                    dma_start(idx1v, t + 2, slot, NB)
            return 0
        lax.fori_loop(0, T // 2, p1_body, 0)

        dma_start(idx2v, 0, 0, CH)
        dma_start(idx2v, 1, 1, CH)
        dma_start(idx2v, 2, 2, CH)

        def p2_chunk(g, slot):
            dma_wait(idx2v, g, slot, CH)
            b = g // T
            t = g - b * T
            evs = tuple(ev[b, t, pl.ds(q * 16, 16)] for q in range(NQ))

            def jstep(j, _):
                accs = mat_gelu(slot, j, evs, t)
                store_scalar(env, b, t * K + 1 + j, sumsq(accs))
                return 0
            lax.fori_loop(0, CH, jstep, 0)

        def p2_body(g3, _):
            for slot in range(3):
                g = g3 * 3 + slot
                p2_chunk(g, slot)

                @pl.when(g + 3 < NCH)
                def _():
                    dma_start(idx2v, g + 3, slot, CH)
            return 0
        lax.fori_loop(0, (NCH - 1) // 3, p2_body, 0)
        p2_chunk(NCH - 1, (NCH - 1) % 3)

        pltpu.sync_copy(env, out_h.at[pl.ds(b0, NB)])

    return sc_energies


def _loss_tc(sumsq2, BT):
    def body(x_ref, o_ref):
        e = jnp.sqrt(x_ref[...])
        m = jnp.max(e, axis=1, keepdims=True)
        lse = m[:, 0] + jnp.log(jnp.sum(jnp.exp(e - m), axis=1))
        o_ref[0, 0] = jnp.sum(lse - e[:, 0]) / _F32(BT)

    out = pl.pallas_call(
        body,
        out_shape=jax.ShapeDtypeStruct((1, 1), _F32),
        out_specs=pl.BlockSpec(memory_space=pltpu.MemorySpace.SMEM),
    )(sumsq2)
    return out[0, 0]


@jax.jit
def kernel(neighbor_ids, weights, biases, node_bias, positions):
    B, T, K, _ = neighbor_ids.shape
    P, H, _ = weights.shape
    V = node_bias.shape[0]

    src = neighbor_ids[..., 0].reshape(B, T * K).astype(_I32)
    tgt = neighbor_ids[..., 1].reshape(B, T * K).astype(_I32)
    w2d = weights.reshape(P, H * H)
    b2d = biases.reshape(P, H)
    nb2d = node_bias.reshape(V, H)
    pe = _pe_table(T + 1, H)

    pos = positions[0, :T, 0]
    ew = jnp.exp(pos - jnp.max(pos))
    s = jnp.concatenate([jnp.zeros((1,), _F32), jnp.cumsum(ew)[:-1]])
    inv_s = jnp.where(jnp.arange(T) > 0, 1.0 / jnp.maximum(s, 1e-30), 0.0)
    ewt = jnp.stack([ew, inv_s]).astype(_F32)

    sc = _make_sc_energies(B, T, K, H, P, V)
    sumsq = sc(src, tgt, w2d, b2d, nb2d, pe, ewt)
    return _loss_tc(sumsq.reshape(B * T, K), B * T)
